# baseline (speedup 1.0000x reference)
import jax
import jax.numpy as jnp
from jax.experimental import pallas as pl
from jax.experimental.pallas import tpu as pltpu

_E = 64
_TOPK = 8
_D = 1024
_I = 512
_IS = _I
_T = 64


def _prologue_kernel(x_ref, gate_w_ref, wsgu_ref, bs_ref, wsd_ref,
                     xt_ref, wts_ref, acc0_ref):
    xt = x_ref[...].T
    xt_ref[...] = xt
    logits = jax.lax.dot(gate_w_ref[...], xt,
                         preferred_element_type=jnp.float32)
    m = jnp.max(logits, axis=0, keepdims=True)
    p = jnp.exp(logits - m)
    p = p / jnp.sum(p, axis=0, keepdims=True)
    iota = jax.lax.broadcasted_iota(jnp.int32, p.shape, 0)
    work = p
    mask = jnp.zeros(p.shape, jnp.float32)
    for _ in range(_TOPK):
        mx = jnp.max(work, axis=0, keepdims=True)
        eq = work == mx
        first = jnp.min(jnp.where(eq, iota, _E), axis=0, keepdims=True)
        pick = iota == first
        mask = mask + pick.astype(jnp.float32)
        work = jnp.where(pick, -1.0, work)
    sel = p * mask
    wts_ref[...] = sel / jnp.sum(sel, axis=0, keepdims=True)
    gus = jax.lax.dot(wsgu_ref[...], xt,
                      preferred_element_type=jnp.float32) + bs_ref[...]
    g, u = gus[:_IS], gus[_IS:]
    acts = g * jax.nn.sigmoid(g) * u
    acc0_ref[...] = jax.lax.dot(wsd_ref[...], acts,
                                preferred_element_type=jnp.float32)


_K = 4
_CGU = 2
_CWD = 1


def _experts_kernel(xt_ref, wts_ref, acc0_ref, wgu_hbm, bgu_ref, wd_hbm,
                    out_ref, acc_ref, bufgu, bufwd, sgu, swd):
    e = pl.program_id(0)

    def issue(ee, s):
        for c in range(_CGU):
            pltpu.make_async_copy(
                wgu_hbm.at[ee, pl.ds(c * (2 * _I // _CGU), 2 * _I // _CGU), :],
                bufgu.at[s, pl.ds(c * (2 * _I // _CGU), 2 * _I // _CGU), :],
                sgu.at[s, c]).start()
        for c in range(_CWD):
            pltpu.make_async_copy(
                wd_hbm.at[ee, pl.ds(c * (_D // _CWD), _D // _CWD), :],
                bufwd.at[s, pl.ds(c * (_D // _CWD), _D // _CWD), :],
                swd.at[s, c]).start()

    @pl.when(e == 0)
    def _init():
        acc_ref[...] = acc0_ref[...]
        for k in range(_K - 1):
            issue(k, k)

    @pl.when(e + _K - 1 < _E)
    def _prefetch():
        ee = e + _K - 1
        issue(ee, jax.lax.rem(ee, _K))

    s = jax.lax.rem(e, _K)
    for c in range(_CGU):
        pltpu.make_async_copy(
            wgu_hbm.at[e, pl.ds(c * (2 * _I // _CGU), 2 * _I // _CGU), :],
            bufgu.at[s, pl.ds(c * (2 * _I // _CGU), 2 * _I // _CGU), :],
            sgu.at[s, c]).wait()
    for c in range(_CWD):
        pltpu.make_async_copy(
            wd_hbm.at[e, pl.ds(c * (_D // _CWD), _D // _CWD), :],
            bufwd.at[s, pl.ds(c * (_D // _CWD), _D // _CWD), :],
            swd.at[s, c]).wait()

    gu = jax.lax.dot(bufgu[s], xt_ref[...],
                     preferred_element_type=jnp.float32) + bgu_ref[0]
    g, u = gu[:_I], gu[_I:]
    w_row = wts_ref[pl.ds(e, 1), :]
    act = g * jax.nn.sigmoid(g) * u * w_row
    acc_ref[...] += jax.lax.dot(bufwd[s], act,
                                preferred_element_type=jnp.float32)

    @pl.when(e == _E - 1)
    def _epilogue():
        out_ref[...] = acc_ref[...].T


def kernel(hidden_states, gate_w, w_gate_up, b_gate_up, w_down,
           ws_gate_up, bs_gate_up, ws_down):
    bgu = b_gate_up.reshape(_E, 2 * _I, 1)
    bs = bs_gate_up.reshape(2 * _IS, 1)

    xt_b, wts, acc0 = pl.pallas_call(
        _prologue_kernel,
        in_specs=[
            pl.BlockSpec((_T, _D), lambda: (0, 0)),
            pl.BlockSpec((_E, _D), lambda: (0, 0)),
            pl.BlockSpec((2 * _IS, _D), lambda: (0, 0)),
            pl.BlockSpec((2 * _IS, 1), lambda: (0, 0)),
            pl.BlockSpec((_D, _IS), lambda: (0, 0)),
        ],
        out_specs=[
            pl.BlockSpec((_D, _T), lambda: (0, 0)),
            pl.BlockSpec((_E, _T), lambda: (0, 0)),
            pl.BlockSpec((_D, _T), lambda: (0, 0)),
        ],
        out_shape=[
            jax.ShapeDtypeStruct((_D, _T), jnp.float32),
            jax.ShapeDtypeStruct((_E, _T), jnp.float32),
            jax.ShapeDtypeStruct((_D, _T), jnp.float32),
        ],
    )(hidden_states, gate_w, ws_gate_up, bs, ws_down)

    out = pl.pallas_call(
        _experts_kernel,
        grid=(_E,),
        in_specs=[
            pl.BlockSpec((_D, _T), lambda e: (0, 0)),
            pl.BlockSpec((_E, _T), lambda e: (0, 0)),
            pl.BlockSpec((_D, _T), lambda e: (0, 0)),
            pl.BlockSpec(memory_space=pltpu.MemorySpace.HBM),
            pl.BlockSpec((1, 2 * _I, 1), lambda e: (e, 0, 0)),
            pl.BlockSpec(memory_space=pltpu.MemorySpace.HBM),
        ],
        out_specs=pl.BlockSpec((_T, _D), lambda e: (0, 0)),
        out_shape=jax.ShapeDtypeStruct((_T, _D), jnp.float32),
        scratch_shapes=[
            pltpu.VMEM((_D, _T), jnp.float32),
            pltpu.VMEM((_K, 2 * _I, _D), jnp.float32),
            pltpu.VMEM((_K, _D, _I), jnp.float32),
            pltpu.SemaphoreType.DMA((_K, _CGU)),
            pltpu.SemaphoreType.DMA((_K, _CWD)),
        ],
        compiler_params=pltpu.CompilerParams(
            dimension_semantics=("arbitrary",),
        ),
    )(xt_b, wts, acc0, w_gate_up, bgu, w_down)
    return out


# speedup vs baseline: 1.0301x; 1.0301x over previous
import jax
import jax.numpy as jnp
from jax.experimental import pallas as pl
from jax.experimental.pallas import tpu as pltpu

_E = 64
_TOPK = 8
_D = 1024
_I = 512
_IS = _I
_T = 64

_K = 4
_CGU = 2
_CWD = 1


def _moe_kernel(x_ref, gate_w_ref, wsgu_ref, bs_ref, wsd_ref,
                wgu_hbm, bgu_ref, wd_hbm,
                out_ref, xt_ref, wts_ref, acc_ref, bufgu, bufwd, sgu, swd):
    e = pl.program_id(0)

    def issue(ee, s):
        for c in range(_CGU):
            pltpu.make_async_copy(
                wgu_hbm.at[ee, pl.ds(c * (2 * _I // _CGU), 2 * _I // _CGU), :],
                bufgu.at[s, pl.ds(c * (2 * _I // _CGU), 2 * _I // _CGU), :],
                sgu.at[s, c]).start()
        for c in range(_CWD):
            pltpu.make_async_copy(
                wd_hbm.at[ee, pl.ds(c * (_D // _CWD), _D // _CWD), :],
                bufwd.at[s, pl.ds(c * (_D // _CWD), _D // _CWD), :],
                swd.at[s, c]).start()

    @pl.when(e == 0)
    def _prologue():
        for k in range(_K - 1):
            issue(k, k)
        xt = x_ref[...].T
        xt_ref[...] = xt
        logits = jax.lax.dot(gate_w_ref[...], xt,
                             preferred_element_type=jnp.float32)
        m = jnp.max(logits, axis=0, keepdims=True)
        p = jnp.exp(logits - m)
        p = p / jnp.sum(p, axis=0, keepdims=True)
        iota = jax.lax.broadcasted_iota(jnp.int32, p.shape, 0)
        work = p
        mask = jnp.zeros(p.shape, jnp.float32)
        for _ in range(_TOPK):
            mx = jnp.max(work, axis=0, keepdims=True)
            eq = work == mx
            first = jnp.min(jnp.where(eq, iota, _E), axis=0, keepdims=True)
            pick = iota == first
            mask = mask + pick.astype(jnp.float32)
            work = jnp.where(pick, -1.0, work)
        sel = p * mask
        wts_ref[...] = sel / jnp.sum(sel, axis=0, keepdims=True)
        gus = jax.lax.dot(wsgu_ref[...], xt,
                          preferred_element_type=jnp.float32) + bs_ref[...]
        g, u = gus[:_IS], gus[_IS:]
        acts = g * jax.nn.sigmoid(g) * u
        acc_ref[...] = jax.lax.dot(wsd_ref[...], acts,
                                   preferred_element_type=jnp.float32)

    @pl.when(e + _K - 1 < _E)
    def _prefetch():
        ee = e + _K - 1
        issue(ee, jax.lax.rem(ee, _K))

    s = jax.lax.rem(e, _K)
    for c in range(_CGU):
        pltpu.make_async_copy(
            wgu_hbm.at[e, pl.ds(c * (2 * _I // _CGU), 2 * _I // _CGU), :],
            bufgu.at[s, pl.ds(c * (2 * _I // _CGU), 2 * _I // _CGU), :],
            sgu.at[s, c]).wait()
    for c in range(_CWD):
        pltpu.make_async_copy(
            wd_hbm.at[e, pl.ds(c * (_D // _CWD), _D // _CWD), :],
            bufwd.at[s, pl.ds(c * (_D // _CWD), _D // _CWD), :],
            swd.at[s, c]).wait()

    gu = jax.lax.dot(bufgu[s], xt_ref[...],
                     preferred_element_type=jnp.float32) + bgu_ref[0]
    g, u = gu[:_I], gu[_I:]
    w_row = wts_ref[pl.ds(e, 1), :]
    act = g * jax.nn.sigmoid(g) * u * w_row
    acc_ref[...] += jax.lax.dot(bufwd[s], act,
                                preferred_element_type=jnp.float32)

    @pl.when(e == _E - 1)
    def _epilogue():
        out_ref[...] = acc_ref[...].T


def kernel(hidden_states, gate_w, w_gate_up, b_gate_up, w_down,
           ws_gate_up, bs_gate_up, ws_down):
    bgu = b_gate_up.reshape(_E, 2 * _I, 1)
    bs = bs_gate_up.reshape(2 * _IS, 1)

    out = pl.pallas_call(
        _moe_kernel,
        grid=(_E,),
        in_specs=[
            pl.BlockSpec((_T, _D), lambda e: (0, 0)),
            pl.BlockSpec((_E, _D), lambda e: (0, 0)),
            pl.BlockSpec((2 * _IS, _D), lambda e: (0, 0)),
            pl.BlockSpec((2 * _IS, 1), lambda e: (0, 0)),
            pl.BlockSpec((_D, _IS), lambda e: (0, 0)),
            pl.BlockSpec(memory_space=pltpu.MemorySpace.HBM),
            pl.BlockSpec((1, 2 * _I, 1), lambda e: (e, 0, 0)),
            pl.BlockSpec(memory_space=pltpu.MemorySpace.HBM),
        ],
        out_specs=pl.BlockSpec((_T, _D), lambda e: (0, 0)),
        out_shape=jax.ShapeDtypeStruct((_T, _D), jnp.float32),
        scratch_shapes=[
            pltpu.VMEM((_D, _T), jnp.float32),
            pltpu.VMEM((_E, _T), jnp.float32),
            pltpu.VMEM((_D, _T), jnp.float32),
            pltpu.VMEM((_K, 2 * _I, _D), jnp.float32),
            pltpu.VMEM((_K, _D, _I), jnp.float32),
            pltpu.SemaphoreType.DMA((_K, _CGU)),
            pltpu.SemaphoreType.DMA((_K, _CWD)),
        ],
        compiler_params=pltpu.CompilerParams(
            dimension_semantics=("arbitrary",),
        ),
    )(hidden_states, gate_w, ws_gate_up, bs, ws_down, w_gate_up, bgu, w_down)
    return out
